# baseline (speedup 1.0000x reference)
import functools

import jax
import jax.numpy as jnp
from jax import lax
from jax.experimental import pallas as pl
from jax.experimental.pallas import tpu as pltpu
from jax.experimental.pallas import tpu_sc as plsc

R, H, K, C = 8, 8, 4096, 128
HC = H * C
NC, NS, L = 2, 16, 16
NW = NC * NS


def kernel(x, prescale, codebook):
    B, S, _ = prescale.shape
    N = B * S
    TW = N // NW
    T = 1
    ROWS = T * R * H
    NCH = TW // T
    NB = 8
    GO = 4

    table = codebook.reshape(R * H * K, C)
    ps = prescale.reshape(N)

    mesh = plsc.VectorSubcoreMesh(
        core_axis_name="c", subcore_axis_name="s",
        num_cores=NC, num_subcores=NS)

    @functools.partial(
        pl.kernel,
        out_type=jax.ShapeDtypeStruct((N, HC), jnp.float32),
        mesh=mesh,
        compiler_params=pltpu.CompilerParams(needs_layout_passes=False),
        scratch_types=[
            pltpu.VMEM((R, H, TW), jnp.int32),
            pltpu.VMEM((TW + L,), jnp.float32),
            pltpu.VMEM((TW * R * H,), jnp.int32),
            pltpu.VMEM((NB, ROWS, C), jnp.float32),
            pltpu.VMEM((2, GO, HC), jnp.float32),
            pltpu.SemaphoreType.DMA((NB,)),
            pltpu.SemaphoreType.DMA((2,)),
        ],
    )
    def sc_kernel(x_hbm, ps_hbm, table_hbm, out_hbm, xv, psv, fidx, buf, ostage,
                  gsem, osem):
        wid = lax.axis_index("s") * NC + lax.axis_index("c")
        base = wid * TW
        pltpu.sync_copy(x_hbm.at[:, :, pl.ds(base, TW)], xv)
        pltpu.sync_copy(ps_hbm.at[pl.ds(base, TW)], psv.at[pl.ds(0, TW)])

        iota = lax.iota(jnp.int32, L)

        def build_slice(s, _):
            pos_base = (iota + s * L) * (R * H)
            for cidx in range(R * H):
                r, h = cidx // H, cidx % H
                vals = xv[r, h, pl.ds(s * L, L)] + cidx * K
                plsc.store_scatter(fidx, [pos_base + h * R + r], vals)
            return 0
        lax.fori_loop(0, TW // L, build_slice, 0)

        def fire_gather(ch, b):
            pltpu.async_copy(
                table_hbm.at[fidx.at[pl.ds(ch * ROWS, ROWS)]], buf.at[b], gsem.at[b])

        def wait_gather(ch, b):
            pltpu.make_async_copy(
                table_hbm.at[fidx.at[pl.ds(ch * ROWS, ROWS)]], buf.at[b], gsem.at[b]).wait()

        def wait_out(g):
            pltpu.make_async_copy(
                ostage.at[g], out_hbm.at[pl.ds(base, GO), :], osem.at[g]).wait()

        for b in range(NB):
            fire_gather(b, b)

        NSL = C // L

        def outer(cg, _):
            for b in range(NB):
                ch = cg * NB + b
                g, jj = b // GO, b % GO
                wait_gather(ch, b)
                if jj == 0:
                    @pl.when(cg >= 1)
                    def _():
                        wait_out(g)
                pvec = psv[pl.ds(ch * T, L)]
                pv = pvec[0]

                @plsc.parallel_loop(0, H * NSL, 1, unroll=2)
                def slice_body(i):
                    h = i // NSL
                    cc = i - h * NSL
                    row0 = h * R
                    sl = pl.ds(cc * L, L)
                    v = [buf[b, row0 + rr, sl] for rr in range(R)]
                    s0 = v[0] + v[1]
                    s1 = v[2] + v[3]
                    s2 = v[4] + v[5]
                    s3 = v[6] + v[7]
                    acc = (s0 + s1) + (s2 + s3)
                    ostage[g, jj, pl.ds(h * C + cc * L, L)] = acc * pv
                if jj == GO - 1:
                    pltpu.async_copy(
                        ostage.at[g],
                        out_hbm.at[pl.ds(base + (ch - (GO - 1)) * T, GO), :],
                        osem.at[g])
                @pl.when(ch + NB < NCH)
                def _():
                    fire_gather(ch + NB, b)
            return 0
        lax.fori_loop(0, NCH // NB, outer, 0)
        for g in range(2):
            wait_out(g)

    out = sc_kernel(x, ps, table)
    return out.reshape(B, S, HC)


# speedup vs baseline: 1.1243x; 1.1243x over previous
import functools

import jax
import jax.numpy as jnp
from jax import lax
from jax.experimental import pallas as pl
from jax.experimental.pallas import tpu as pltpu
from jax.experimental.pallas import tpu_sc as plsc

R, H, K, C = 8, 8, 4096, 128
HC = H * C
NC, NS, L = 2, 16, 16
NW = NC * NS


def kernel(x, prescale, codebook):
    B, S, _ = prescale.shape
    N = B * S
    TW = N // NW
    T = 1
    ROWS = T * R * H
    NCH = TW // T
    NB = 8

    table = codebook.reshape(R * H * K, C)
    ps = prescale.reshape(N)

    mesh = plsc.VectorSubcoreMesh(
        core_axis_name="c", subcore_axis_name="s",
        num_cores=NC, num_subcores=NS)

    @functools.partial(
        pl.kernel,
        out_type=jax.ShapeDtypeStruct((N, HC), jnp.float32),
        mesh=mesh,
        compiler_params=pltpu.CompilerParams(needs_layout_passes=False),
        scratch_types=[
            pltpu.VMEM((R, H, TW), jnp.int32),
            pltpu.VMEM((TW + L,), jnp.float32),
            pltpu.VMEM((TW * R * H,), jnp.int32),
            pltpu.VMEM((NB, ROWS, C), jnp.float32),
            pltpu.VMEM((NB, T, HC), jnp.float32),
            pltpu.SemaphoreType.DMA((NB,)),
            pltpu.SemaphoreType.DMA((NB,)),
        ],
    )
    def sc_kernel(x_hbm, ps_hbm, table_hbm, out_hbm, xv, psv, fidx, buf, ostage,
                  gsem, osem):
        wid = lax.axis_index("s") * NC + lax.axis_index("c")
        base = wid * TW
        pltpu.sync_copy(x_hbm.at[:, :, pl.ds(base, TW)], xv)
        pltpu.sync_copy(ps_hbm.at[pl.ds(base, TW)], psv.at[pl.ds(0, TW)])

        iota = lax.iota(jnp.int32, L)

        def build_slice(s, _):
            pos_base = (iota + s * L) * (R * H)
            for cidx in range(R * H):
                r, h = cidx // H, cidx % H
                vals = xv[r, h, pl.ds(s * L, L)] + cidx * K
                plsc.store_scatter(fidx, [pos_base + h * R + r], vals)
            return 0

        def fire_gather(ch, b):
            pltpu.async_copy(
                table_hbm.at[fidx.at[pl.ds(ch * ROWS, ROWS)]], buf.at[b], gsem.at[b])

        def wait_gather(ch, b):
            pltpu.make_async_copy(
                table_hbm.at[fidx.at[pl.ds(ch * ROWS, ROWS)]], buf.at[b], gsem.at[b]).wait()

        def wait_out(b):
            pltpu.make_async_copy(
                ostage.at[b], out_hbm.at[pl.ds(base, T), :], osem.at[b]).wait()

        build_slice(0, 0)
        for b in range(NB):
            fire_gather(b, b)
        lax.fori_loop(1, TW // L, build_slice, 0)

        def outer(cg, _):
            for b in range(NB):
                ch = cg * NB + b
                wait_gather(ch, b)
                @pl.when(ch >= NB)
                def _():
                    wait_out(b)
                pvec = psv[pl.ds(ch * T, L)]
                NSL = C // L
                for j in range(T):
                    pv = pvec[j]

                    @plsc.parallel_loop(0, H * NSL, 1, unroll=2)
                    def slice_body(i):
                        h = i // NSL
                        cc = i - h * NSL
                        row0 = (j * H + h) * R
                        sl = pl.ds(cc * L, L)
                        v = [buf[b, row0 + rr, sl] for rr in range(R)]
                        s0 = v[0] + v[1]
                        s1 = v[2] + v[3]
                        s2 = v[4] + v[5]
                        s3 = v[6] + v[7]
                        acc = (s0 + s1) + (s2 + s3)
                        ostage[b, j, pl.ds(h * C + cc * L, L)] = acc * pv
                pltpu.async_copy(
                    ostage.at[b], out_hbm.at[pl.ds(base + ch * T, T), :], osem.at[b])
                @pl.when(ch + NB < NCH)
                def _():
                    fire_gather(ch + NB, b)
            return 0
        lax.fori_loop(0, NCH // NB, outer, 0)
        for b in range(NB):
            wait_out(b)

    out = sc_kernel(x, ps, table)
    return out.reshape(B, S, HC)


# speedup vs baseline: 1.1300x; 1.0051x over previous
import functools

import jax
import jax.numpy as jnp
from jax import lax
from jax.experimental import pallas as pl
from jax.experimental.pallas import tpu as pltpu
from jax.experimental.pallas import tpu_sc as plsc

R, H, K, C = 8, 8, 4096, 128
HC = H * C
NC, NS, L = 2, 16, 16
NW = NC * NS


def kernel(x, prescale, codebook):
    B, S, _ = prescale.shape
    N = B * S
    TW = N // NW
    T = 1
    ROWS = T * R * H
    NCH = TW // T
    NB = 8

    table = codebook.reshape(R * H * K, C)
    ps = prescale.reshape(N)

    mesh = plsc.VectorSubcoreMesh(
        core_axis_name="c", subcore_axis_name="s",
        num_cores=NC, num_subcores=NS)

    @functools.partial(
        pl.kernel,
        out_type=jax.ShapeDtypeStruct((N, HC), jnp.float32),
        mesh=mesh,
        compiler_params=pltpu.CompilerParams(needs_layout_passes=False),
        scratch_types=[
            pltpu.VMEM((R, H, TW), jnp.int32),
            pltpu.VMEM((TW + L,), jnp.float32),
            pltpu.VMEM((TW * R * H,), jnp.int32),
            pltpu.VMEM((NB, ROWS, C), jnp.float32),
            pltpu.VMEM((NB, T, HC), jnp.float32),
            pltpu.SemaphoreType.DMA((NB,)),
            pltpu.SemaphoreType.DMA((NB,)),
        ],
    )
    def sc_kernel(x_hbm, ps_hbm, table_hbm, out_hbm, xv, psv, fidx, buf, ostage,
                  gsem, osem):
        wid = lax.axis_index("s") * NC + lax.axis_index("c")
        base = wid * TW
        pltpu.sync_copy(x_hbm.at[:, :, pl.ds(base, TW)], xv)
        pltpu.sync_copy(ps_hbm.at[pl.ds(base, TW)], psv.at[pl.ds(0, TW)])

        iota = lax.iota(jnp.int32, L)

        def build_slice(s, _):
            pos_base = (iota + s * L) * (R * H)
            for cidx in range(R * H):
                r, h = cidx // H, cidx % H
                vals = xv[r, h, pl.ds(s * L, L)] + cidx * K
                plsc.store_scatter(fidx, [pos_base + h * R + r], vals)
            return 0

        def fire_gather(ch, b):
            pltpu.async_copy(
                table_hbm.at[fidx.at[pl.ds(ch * ROWS, ROWS)]], buf.at[b], gsem.at[b])

        def wait_gather(ch, b):
            pltpu.make_async_copy(
                table_hbm.at[fidx.at[pl.ds(ch * ROWS, ROWS)]], buf.at[b], gsem.at[b]).wait()

        def wait_out(b):
            pltpu.make_async_copy(
                ostage.at[b], out_hbm.at[pl.ds(base, T), :], osem.at[b]).wait()

        build_slice(0, 0)
        for b in range(NB):
            fire_gather(b, b)

        @plsc.parallel_loop(1, TW // L, 1, unroll=2)
        def _build_rest(s):
            build_slice(s, 0)

        def outer(cg, _):
            for b in range(NB):
                ch = cg * NB + b
                wait_gather(ch, b)
                @pl.when(ch >= NB)
                def _():
                    wait_out(b)
                pvec = psv[pl.ds(ch * T, L)]
                NSL = C // L
                for j in range(T):
                    pv = pvec[j]

                    @plsc.parallel_loop(0, H * NSL, 1, unroll=2)
                    def slice_body(i):
                        h = i // NSL
                        cc = i - h * NSL
                        row0 = (j * H + h) * R
                        sl = pl.ds(cc * L, L)
                        v = [buf[b, row0 + rr, sl] for rr in range(R)]
                        s0 = v[0] + v[1]
                        s1 = v[2] + v[3]
                        s2 = v[4] + v[5]
                        s3 = v[6] + v[7]
                        acc = (s0 + s1) + (s2 + s3)
                        ostage[b, j, pl.ds(h * C + cc * L, L)] = acc * pv
                pltpu.async_copy(
                    ostage.at[b], out_hbm.at[pl.ds(base + ch * T, T), :], osem.at[b])
                @pl.when(ch + NB < NCH)
                def _():
                    fire_gather(ch + NB, b)
            return 0
        lax.fori_loop(0, NCH // NB, outer, 0)
        for b in range(NB):
            wait_out(b)

    out = sc_kernel(x, ps, table)
    return out.reshape(B, S, HC)


# speedup vs baseline: 1.1335x; 1.0031x over previous
import functools

import jax
import jax.numpy as jnp
from jax import lax
from jax.experimental import pallas as pl
from jax.experimental.pallas import tpu as pltpu
from jax.experimental.pallas import tpu_sc as plsc

R, H, K, C = 8, 8, 4096, 128
HC = H * C
NC, NS, L = 2, 16, 16
NW = NC * NS


def kernel(x, prescale, codebook):
    B, S, _ = prescale.shape
    N = B * S
    TW = N // NW
    T = 1
    ROWS = T * R * H
    NCH = TW // T
    NB = 8

    table = codebook.reshape(R * H * K, C)
    ps = prescale.reshape(N)

    mesh = plsc.VectorSubcoreMesh(
        core_axis_name="c", subcore_axis_name="s",
        num_cores=NC, num_subcores=NS)

    @functools.partial(
        pl.kernel,
        out_type=jax.ShapeDtypeStruct((N, HC), jnp.float32),
        mesh=mesh,
        compiler_params=pltpu.CompilerParams(needs_layout_passes=False),
        scratch_types=[
            pltpu.VMEM((R, H, TW), jnp.int32),
            pltpu.VMEM((TW + L,), jnp.float32),
            pltpu.VMEM((TW * R * H,), jnp.int32),
            pltpu.VMEM((NB, ROWS, C), jnp.float32),
            pltpu.VMEM((NB, T, HC), jnp.float32),
            pltpu.SemaphoreType.DMA((NB,)),
            pltpu.SemaphoreType.DMA((NB,)),
        ],
    )
    def sc_kernel(x_hbm, ps_hbm, table_hbm, out_hbm, xv, psv, fidx, buf, ostage,
                  gsem, osem):
        wid = lax.axis_index("s") * NC + lax.axis_index("c")
        base = wid * TW
        pltpu.sync_copy(x_hbm.at[:, :, pl.ds(base, TW)], xv)
        pltpu.sync_copy(ps_hbm.at[pl.ds(base, TW)], psv.at[pl.ds(0, TW)])

        iota = lax.iota(jnp.int32, L)

        def build_slice(s, _):
            pos_base = (iota + s * L) * (R * H)
            for cidx in range(R * H):
                r, h = cidx // H, cidx % H
                vals = xv[r, h, pl.ds(s * L, L)] + cidx * K
                plsc.store_scatter(fidx, [pos_base + h * R + r], vals)
            return 0

        HR = ROWS // 2

        def fire_gather(ch, b):
            pltpu.async_copy(
                table_hbm.at[fidx.at[pl.ds(ch * ROWS, HR)]],
                buf.at[b, pl.ds(0, HR)], gsem.at[b])
            pltpu.async_copy(
                table_hbm.at[fidx.at[pl.ds(ch * ROWS + HR, HR)]],
                buf.at[b, pl.ds(HR, HR)], gsem.at[b])

        def wait_gather(ch, b):
            pltpu.make_async_copy(
                table_hbm.at[fidx.at[pl.ds(ch * ROWS, HR)]],
                buf.at[b, pl.ds(0, HR)], gsem.at[b]).wait()
            pltpu.make_async_copy(
                table_hbm.at[fidx.at[pl.ds(ch * ROWS + HR, HR)]],
                buf.at[b, pl.ds(HR, HR)], gsem.at[b]).wait()

        def wait_out(b):
            pltpu.make_async_copy(
                ostage.at[b], out_hbm.at[pl.ds(base, T), :], osem.at[b]).wait()

        build_slice(0, 0)
        for b in range(NB):
            fire_gather(b, b)

        @plsc.parallel_loop(1, TW // L, 1, unroll=2)
        def _build_rest(s):
            build_slice(s, 0)

        def outer(cg, _):
            for b in range(NB):
                ch = cg * NB + b
                wait_gather(ch, b)
                @pl.when(ch >= NB)
                def _():
                    wait_out(b)
                pvec = psv[pl.ds(ch * T, L)]
                NSL = C // L
                for j in range(T):
                    pv = pvec[j]

                    @plsc.parallel_loop(0, H * NSL, 1, unroll=2)
                    def slice_body(i):
                        h = i // NSL
                        cc = i - h * NSL
                        row0 = (j * H + h) * R
                        sl = pl.ds(cc * L, L)
                        v = [buf[b, row0 + rr, sl] for rr in range(R)]
                        s0 = v[0] + v[1]
                        s1 = v[2] + v[3]
                        s2 = v[4] + v[5]
                        s3 = v[6] + v[7]
                        acc = (s0 + s1) + (s2 + s3)
                        ostage[b, j, pl.ds(h * C + cc * L, L)] = acc * pv
                pltpu.async_copy(
                    ostage.at[b], out_hbm.at[pl.ds(base + ch * T, T), :], osem.at[b])
                @pl.when(ch + NB < NCH)
                def _():
                    fire_gather(ch + NB, b)
            return 0
        lax.fori_loop(0, NCH // NB, outer, 0)
        for b in range(NB):
            wait_out(b)

    out = sc_kernel(x, ps, table)
    return out.reshape(B, S, HC)
